# baseline (speedup 1.0000x reference)
import functools

import jax
import jax.numpy as jnp
from jax import lax
from jax.experimental import pallas as pl
from jax.experimental.pallas import tpu as pltpu
from jax.experimental.pallas import tpu_sc as plsc

_SEQ = 2048
_DIM = 768
_SCALE = 8.0

_info = plsc.get_sparse_core_info()
_NC, _NS = _info.num_cores, _info.num_subcores
_NW = _NC * _NS
_B_PER_W = _SEQ // _NW

_mesh = plsc.VectorSubcoreMesh(core_axis_name="c", subcore_axis_name="s")


def _make_gather(seq):
    b_per_w = seq // _NW

    @functools.partial(
        pl.kernel,
        mesh=_mesh,
        out_type=jax.ShapeDtypeStruct((seq, _DIM), jnp.float32),
        scratch_types=[
            pltpu.VMEM((b_per_w,), jnp.int32),
            pltpu.VMEM((b_per_w, _DIM), jnp.float32),
            pltpu.SemaphoreType.DMA,
        ],
    )
    def _emb_gather(table_hbm, ids_hbm, out_hbm, idx_v, rows_v, sem):
        wid = lax.axis_index("s") * _NC + lax.axis_index("c")
        base = wid * b_per_w
        pltpu.sync_copy(ids_hbm.at[pl.ds(base, b_per_w)], idx_v)
        pltpu.async_copy(table_hbm.at[idx_v], rows_v, sem).wait()

        def scale_row(i, _):
            for j in range(_DIM // 16):
                sl = pl.ds(j * 16, 16)
                rows_v[i, sl] = rows_v[i, sl] * _SCALE
            return _

        lax.fori_loop(0, b_per_w, scale_row, None)
        pltpu.sync_copy(rows_v, out_hbm.at[pl.ds(base, b_per_w)])

    return _emb_gather


_gather_half = _make_gather(_SEQ // 2)


def kernel(token_ids, per_layer_table):
    b, s = token_ids.shape
    ids = token_ids.reshape(-1).astype(jnp.int32)
    h = s // 2
    o1 = _gather_half(per_layer_table, ids[:h])
    o2 = _gather_half(per_layer_table, ids[h:])
    return jnp.concatenate(
        [o1.reshape(b, h, 12, 64), o2.reshape(b, h, 12, 64)], axis=1
    )


# speedup vs baseline: 1.2362x; 1.2362x over previous
import functools

import jax
import jax.numpy as jnp
from jax import lax
from jax.experimental import pallas as pl
from jax.experimental.pallas import tpu as pltpu
from jax.experimental.pallas import tpu_sc as plsc

_SEQ = 2048
_DIM = 768
_SCALE = 8.0

_info = plsc.get_sparse_core_info()
_NC, _NS = _info.num_cores, _info.num_subcores
_NW = _NC * _NS
_B_PER_W = _SEQ // _NW
_NB = 4
_BLK = _B_PER_W // _NB

_mesh = plsc.VectorSubcoreMesh(core_axis_name="c", subcore_axis_name="s")


@functools.partial(
    pl.kernel,
    mesh=_mesh,
    out_type=jax.ShapeDtypeStruct((_SEQ, _DIM), jnp.float32),
    scratch_types=[
        pltpu.VMEM((_B_PER_W,), jnp.int32),
        pltpu.VMEM((_B_PER_W, _DIM), jnp.float32),
        [pltpu.SemaphoreType.DMA] * _NB,
        pltpu.SemaphoreType.DMA,
    ],
)
def _emb_gather(table_hbm, ids_hbm, out_hbm, idx_v, rows_v, gsems, osem):
    wid = lax.axis_index("s") * _NC + lax.axis_index("c")
    base = wid * _B_PER_W
    pltpu.sync_copy(ids_hbm.at[pl.ds(base, _B_PER_W)], idx_v)

    gathers = []
    for b in range(_NB):
        blk = pl.ds(b * _BLK, _BLK)
        gathers.append(
            pltpu.async_copy(table_hbm.at[idx_v.at[blk]], rows_v.at[blk], gsems[b])
        )

    writes = []
    for b in range(_NB):
        gathers[b].wait()

        def scale_row(i, _):
            for j in range(_DIM // 16):
                sl = pl.ds(j * 16, 16)
                rows_v[i, sl] = rows_v[i, sl] * _SCALE
            return _

        lax.fori_loop(b * _BLK, (b + 1) * _BLK, scale_row, None)
        blk = pl.ds(b * _BLK, _BLK)
        writes.append(
            pltpu.async_copy(
                rows_v.at[blk], out_hbm.at[pl.ds(base + b * _BLK, _BLK)], osem
            )
        )

    for w in writes:
        w.wait()


def kernel(token_ids, per_layer_table):
    b, s = token_ids.shape
    ids = token_ids.reshape(-1).astype(jnp.int32)
    out = _emb_gather(per_layer_table, ids)
    return out.reshape(b, s, 12, 64)
